# baseline (speedup 1.0000x reference)
import jax
import jax.numpy as jnp
from jax import lax
from jax.experimental import pallas as pl
from jax.experimental.pallas import tpu as pltpu
from jax.experimental.pallas import tpu_sc as plsc

_EMBED = 32
_BATCH = 4096
_HIST = 200

_INFO = plsc.get_sparse_core_info()
_NC = _INFO.num_cores
_NS = _INFO.num_subcores
_NW = _NC * _NS
_BB = _BATCH // _NW
_WP = _BB + 1
_NBUF = 8
_T = _HIST // _NBUF


def _body(idx_hbm, table_hbm, out_hbm, idx_v, gbufs, wbufs, gsem, wsem):
    wid = lax.axis_index("s") * _NC + lax.axis_index("c")
    bbase = wid * _BB
    pltpu.sync_copy(idx_hbm.at[:, pl.ds(bbase, _BB)], idx_v)

    jlo = lax.iota(jnp.int32, 16)
    jhi = jlo + 16

    def fire(b, s):
        pltpu.async_copy(table_hbm.at[idx_v.at[s]], gbufs.at[b], gsem.at[b])

    def wait_gather(b):
        pltpu.make_async_copy(
            table_hbm.at[pl.ds(0, _BB)], gbufs.at[b], gsem.at[b]).wait()

    def transpose(b):
        unroll = 8

        def tr_step(i, carry):
            r0 = i * unroll
            rs = jnp.full((16,), r0, jnp.int32)
            for k in range(unroll):
                r = r0 + k
                rsk = rs + k
                v0 = gbufs[b, r, pl.ds(0, 16)]
                v1 = gbufs[b, r, pl.ds(16, 16)]
                plsc.store_scatter(wbufs.at[b], [jlo, rsk], v0)
                plsc.store_scatter(wbufs.at[b], [jhi, rsk], v1)
            return carry

        lax.fori_loop(0, _BB // unroll, tr_step, 0)

    def start_write(b, s):
        for jt in range(4):
            pltpu.async_copy(
                wbufs.at[b, pl.ds(8 * jt, 8), pl.ds(0, _BB)],
                out_hbm.at[s, jt, wid, :, :],
                wsem.at[b],
            )

    def wait_write(b):
        for jt in range(4):
            pltpu.make_async_copy(
                wbufs.at[b, pl.ds(8 * jt, 8), pl.ds(0, _BB)],
                out_hbm.at[0, 0, 0, :, :],
                wsem.at[b],
            ).wait()

    for b in range(_NBUF - 1):
        fire(b, b)

    def step(t, carry):
        for b in range(_NBUF):
            s = t * _NBUF + b
            wait_gather(b)

            @pl.when(t > 0)
            def _():
                wait_write(b)
            transpose(b)
            start_write(b, s)
            if b == 0:
                fire((b - 1) % _NBUF, s + _NBUF - 1)
            else:
                @pl.when(t < _T - 1)
                def _():
                    fire(b - 1, s + _NBUF - 1)
        return carry

    lax.fori_loop(0, _T, step, 0)
    for b in range(_NBUF):
        wait_write(b)


@jax.jit
def _gather(idx, table):
    mesh = plsc.VectorSubcoreMesh(core_axis_name="c", subcore_axis_name="s")
    f = pl.kernel(
        _body,
        out_type=jax.ShapeDtypeStruct(
            (_HIST, _EMBED // 8, _NW, 8, _BB), jnp.float32),
        mesh=mesh,
        compiler_params=pltpu.CompilerParams(
            use_tc_tiling_on_sc=False, needs_layout_passes=False,
            disable_bounds_checks=True),
        scratch_types=[
            pltpu.VMEM((_HIST, _BB), jnp.int32),
            pltpu.VMEM((_NBUF, _BB, _EMBED), jnp.float32),
            pltpu.VMEM((_NBUF, _EMBED, _WP), jnp.float32),
            pltpu.SemaphoreType.DMA((_NBUF,)),
            pltpu.SemaphoreType.DMA((_NBUF,)),
        ],
    )
    return f(idx, table)


def kernel(tags, table):
    idx = jnp.transpose(tags).astype(jnp.int32)
    out5 = _gather(idx, table)
    out = jnp.transpose(out5, (2, 4, 0, 1, 3)).reshape(_BATCH, _HIST, _EMBED)
    return out


# speedup vs baseline: 1.0019x; 1.0019x over previous
import jax
import jax.numpy as jnp
from jax import lax
from jax.experimental import pallas as pl
from jax.experimental.pallas import tpu as pltpu
from jax.experimental.pallas import tpu_sc as plsc

_EMBED = 32
_BATCH = 4096
_HIST = 200

_INFO = plsc.get_sparse_core_info()
_NC = _INFO.num_cores
_NS = _INFO.num_subcores
_NW = _NC * _NS
_BB = _BATCH // _NW
_WP = _BB + 1
_NBUF = 4
_T = _HIST // _NBUF


def _body(idx_hbm, table_hbm, out_hbm, idx_v, gbufs, wbufs, gsem, wsem):
    wid = lax.axis_index("s") * _NC + lax.axis_index("c")
    bbase = wid * _BB
    pltpu.sync_copy(idx_hbm.at[:, pl.ds(bbase, _BB)], idx_v)

    jlo = lax.iota(jnp.int32, 16)
    jhi = jlo + 16

    def fire(b, s):
        pltpu.async_copy(table_hbm.at[idx_v.at[s]], gbufs.at[b], gsem.at[b])

    def wait_gather(b):
        pltpu.make_async_copy(
            table_hbm.at[pl.ds(0, _BB)], gbufs.at[b], gsem.at[b]).wait()

    def transpose(b):
        unroll = 8

        def tr_step(i, carry):
            r0 = i * unroll
            rs = jnp.full((16,), r0, jnp.int32)
            for k in range(unroll):
                r = r0 + k
                rsk = rs + k
                v0 = gbufs[b, r, pl.ds(0, 16)]
                v1 = gbufs[b, r, pl.ds(16, 16)]
                plsc.store_scatter(wbufs.at[b], [jlo, rsk], v0)
                plsc.store_scatter(wbufs.at[b], [jhi, rsk], v1)
            return carry

        lax.fori_loop(0, _BB // unroll, tr_step, 0)

    def start_write(b, s):
        for jt in range(4):
            pltpu.async_copy(
                wbufs.at[b, pl.ds(8 * jt, 8), pl.ds(0, _BB)],
                out_hbm.at[s, jt, wid, :, :],
                wsem.at[b],
            )

    def wait_write(b):
        for jt in range(4):
            pltpu.make_async_copy(
                wbufs.at[b, pl.ds(8 * jt, 8), pl.ds(0, _BB)],
                out_hbm.at[0, 0, 0, :, :],
                wsem.at[b],
            ).wait()

    for b in range(_NBUF - 1):
        fire(b, b)

    def step(t, carry):
        for b in range(_NBUF):
            s = t * _NBUF + b
            wait_gather(b)

            @pl.when(t > 0)
            def _():
                wait_write(b)
            transpose(b)
            start_write(b, s)
            if b == 0:
                fire((b - 1) % _NBUF, s + _NBUF - 1)
            else:
                @pl.when(t < _T - 1)
                def _():
                    fire(b - 1, s + _NBUF - 1)
        return carry

    lax.fori_loop(0, _T, step, 0)
    for b in range(_NBUF):
        wait_write(b)


@jax.jit
def _gather(idx, table):
    mesh = plsc.VectorSubcoreMesh(core_axis_name="c", subcore_axis_name="s")
    f = pl.kernel(
        _body,
        out_type=jax.ShapeDtypeStruct(
            (_HIST, _EMBED // 8, _NW, 8, _BB), jnp.float32),
        mesh=mesh,
        compiler_params=pltpu.CompilerParams(
            use_tc_tiling_on_sc=False, needs_layout_passes=False,
            disable_bounds_checks=True),
        scratch_types=[
            pltpu.VMEM((_HIST, _BB), jnp.int32),
            pltpu.VMEM((_NBUF, _BB, _EMBED), jnp.float32),
            pltpu.VMEM((_NBUF, _EMBED, _WP), jnp.float32),
            pltpu.SemaphoreType.DMA((_NBUF,)),
            pltpu.SemaphoreType.DMA((_NBUF,)),
        ],
    )
    return f(idx, table)


def kernel(tags, table):
    idx = jnp.transpose(tags).astype(jnp.int32)
    out5 = _gather(idx, table)
    out = jnp.transpose(out5, (2, 4, 0, 1, 3)).reshape(_BATCH, _HIST, _EMBED)
    return out


# speedup vs baseline: 1.0045x; 1.0026x over previous
import jax
import jax.numpy as jnp
from jax import lax
from jax.experimental import pallas as pl
from jax.experimental.pallas import tpu as pltpu
from jax.experimental.pallas import tpu_sc as plsc

_EMBED = 32
_BATCH = 4096
_HIST = 200

_INFO = plsc.get_sparse_core_info()
_NC = _INFO.num_cores
_NS = _INFO.num_subcores
_NW = _NC * _NS
_BB = _BATCH // _NW
_WP = _BB + 1
_NBUF = 4
_T = _HIST // _NBUF


def _body(idx_hbm, table_hbm, out_hbm, idx_v, gbufs, wbufs, gsem, wsem):
    wid = lax.axis_index("s") * _NC + lax.axis_index("c")
    bbase = wid * _BB
    pltpu.sync_copy(idx_hbm.at[:, pl.ds(bbase, _BB)], idx_v)

    jlo = lax.iota(jnp.int32, 16)
    jhi = jlo + 16

    def fire(b, s):
        pltpu.async_copy(table_hbm.at[idx_v.at[s]], gbufs.at[b], gsem.at[b])

    def wait_gather(b):
        pltpu.make_async_copy(
            table_hbm.at[pl.ds(0, _BB)], gbufs.at[b], gsem.at[b]).wait()

    def transpose(b):
        unroll = 16

        def tr_step(i, carry):
            r0 = i * unroll
            rs = jnp.full((16,), r0, jnp.int32)
            for k in range(unroll):
                r = r0 + k
                rsk = rs + k
                v0 = gbufs[b, r, pl.ds(0, 16)]
                v1 = gbufs[b, r, pl.ds(16, 16)]
                plsc.store_scatter(wbufs.at[b], [jlo, rsk], v0)
                plsc.store_scatter(wbufs.at[b], [jhi, rsk], v1)
            return carry

        lax.fori_loop(0, _BB // unroll, tr_step, 0)

    def start_write(b, s):
        for jt in range(4):
            pltpu.async_copy(
                wbufs.at[b, pl.ds(8 * jt, 8), pl.ds(0, _BB)],
                out_hbm.at[s, jt, wid, :, :],
                wsem.at[b],
            )

    def wait_write(b):
        for jt in range(4):
            pltpu.make_async_copy(
                wbufs.at[b, pl.ds(8 * jt, 8), pl.ds(0, _BB)],
                out_hbm.at[0, 0, 0, :, :],
                wsem.at[b],
            ).wait()

    for b in range(_NBUF - 1):
        fire(b, b)

    def step(t, carry):
        for b in range(_NBUF):
            s = t * _NBUF + b
            wait_gather(b)

            @pl.when(t > 0)
            def _():
                wait_write(b)
            transpose(b)
            start_write(b, s)
            if b == 0:
                fire((b - 1) % _NBUF, s + _NBUF - 1)
            else:
                @pl.when(t < _T - 1)
                def _():
                    fire(b - 1, s + _NBUF - 1)
        return carry

    lax.fori_loop(0, _T, step, 0)
    for b in range(_NBUF):
        wait_write(b)


@jax.jit
def _gather(idx, table):
    mesh = plsc.VectorSubcoreMesh(core_axis_name="c", subcore_axis_name="s")
    f = pl.kernel(
        _body,
        out_type=jax.ShapeDtypeStruct(
            (_HIST, _EMBED // 8, _NW, 8, _BB), jnp.float32),
        mesh=mesh,
        compiler_params=pltpu.CompilerParams(
            use_tc_tiling_on_sc=False, needs_layout_passes=False,
            disable_bounds_checks=True),
        scratch_types=[
            pltpu.VMEM((_HIST, _BB), jnp.int32),
            pltpu.VMEM((_NBUF, _BB, _EMBED), jnp.float32),
            pltpu.VMEM((_NBUF, _EMBED, _WP), jnp.float32),
            pltpu.SemaphoreType.DMA((_NBUF,)),
            pltpu.SemaphoreType.DMA((_NBUF,)),
        ],
    )
    return f(idx, table)


def kernel(tags, table):
    idx = jnp.transpose(tags).astype(jnp.int32)
    out5 = _gather(idx, table)
    out = jnp.transpose(out5, (2, 4, 0, 1, 3)).reshape(_BATCH, _HIST, _EMBED)
    return out
